# baseline (speedup 1.0000x reference)
import jax
import jax.numpy as jnp
from jax.experimental import pallas as pl


def _add_block(patches_ref, pos_ref, out_ref):
    out_ref[...] = patches_ref[...] + pos_ref[...]


def kernel(patches, pos_table):
    batch, n_patches, model_dim = patches.shape
    bb = 2
    return pl.pallas_call(
        _add_block,
        grid=(batch // bb,),
        in_specs=[
            pl.BlockSpec((bb, n_patches, model_dim), lambda i: (i, 0, 0)),
            pl.BlockSpec((n_patches, model_dim), lambda i: (0, 0)),
        ],
        out_specs=pl.BlockSpec((bb, n_patches, model_dim), lambda i: (i, 0, 0)),
        out_shape=jax.ShapeDtypeStruct((batch, n_patches, model_dim), patches.dtype),
    )(patches, pos_table)


# speedup vs baseline: 1.0489x; 1.0489x over previous
import jax
import jax.numpy as jnp
from jax.experimental import pallas as pl
from jax.experimental.pallas import tpu as pltpu


def _add_block(patches_ref, pos_ref, out_ref):
    out_ref[...] = patches_ref[...] + pos_ref[...]


def kernel(patches, pos_table):
    batch, n_patches, model_dim = patches.shape
    bb = 8
    return pl.pallas_call(
        _add_block,
        grid=(batch // bb,),
        compiler_params=pltpu.CompilerParams(vmem_limit_bytes=120 * 1024 * 1024),
        in_specs=[
            pl.BlockSpec((bb, n_patches, model_dim), lambda i: (i, 0, 0)),
            pl.BlockSpec((n_patches, model_dim), lambda i: (0, 0)),
        ],
        out_specs=pl.BlockSpec((bb, n_patches, model_dim), lambda i: (i, 0, 0)),
        out_shape=jax.ShapeDtypeStruct((batch, n_patches, model_dim), patches.dtype),
    )(patches, pos_table)
